# baseline (speedup 1.0000x reference)
import functools

import jax
import jax.numpy as jnp
from jax import lax
from jax.experimental import pallas as pl
from jax.experimental.pallas import tpu as pltpu
from jax.experimental.pallas import tpu_sc as plsc

VOCAB = 100000
D = 64
B = 4096
C = 20
NNEG = 20

NC = 2
NS = 16
NW = NC * NS
LANES = 16

WP = D // 2
K = C + C * NNEG
KPAD = 432
NGRP = KPAD // LANES
B_PER_W = B // NW
CB = 16
NCH = B_PER_W // CB

_GATHER_SLICES = ((0, 128), (128, 128), (256, 128), (384, KPAD - 384))


def _sc_scores_kernel(iword_hbm, cwords_hbm, wi_hbm, wo_hbm, out_hbm,
                      iwv, cidx_v, ivecs_v, rows_v, scores_v,
                      sem_i, sem0, sem1, sem2, sem3):
    wid = lax.axis_index("s") * NC + lax.axis_index("c")
    wbase = wid * B_PER_W
    NSLOT = 4
    slot_refs = tuple(rows_v.at[s] for s in range(NSLOT))
    sems = (sem0, sem1, sem2, sem3)

    def issue4(bi, slot):
        ib = bi * KPAD
        for off, ln in _GATHER_SLICES:
            pltpu.async_copy(
                wo_hbm.at[cidx_v.at[pl.ds(ib + off, ln)]],
                slot_refs[slot].at[pl.ds(off, ln)], sems[slot])

    def wait4(bi, slot):
        ib = bi * KPAD
        for off, ln in _GATHER_SLICES:
            pltpu.make_async_copy(
                wo_hbm.at[cidx_v.at[pl.ds(ib + off, ln)]],
                slot_refs[slot].at[pl.ds(off, ln)], sems[slot]).wait()

    def compute(bi, slot):
        ib = bi * KPAD
        iota = lax.iota(jnp.int32, LANES)

        def rot(v, k):
            perm = (iota + k) & (LANES - 1)
            return v.at[perm].get(mode="promise_in_bounds")

        t_e = (2 * iota) & (LANES - 1)
        half = iota < (LANES // 2)
        eo = []
        for dd in range(WP // LANES):
            iv_a = ivecs_v[bi, pl.ds(dd * 2 * LANES, LANES)]
            iv_b = ivecs_v[bi, pl.ds(dd * 2 * LANES + LANES, LANES)]
            e_v = jnp.where(half,
                            iv_a.at[t_e].get(mode="promise_in_bounds"),
                            iv_b.at[t_e].get(mode="promise_in_bounds"))
            o_v = jnp.where(half,
                            iv_a.at[t_e + 1].get(mode="promise_in_bounds"),
                            iv_b.at[t_e + 1].get(mode="promise_in_bounds"))
            eo.append((e_v, o_v))

        def g_step(g, _):
            rid = g * LANES + iota
            zeros = jnp.zeros((LANES,), jnp.float32)
            accs = [zeros, zeros, zeros, zeros]
            for dd in range(WP // LANES):
                e_v, o_v = eo[dd]
                for wu in range(LANES):
                    perm = (iota + wu) & (LANES - 1)
                    word = plsc.load_gather(
                        slot_refs[slot], [rid, dd * LANES + perm])
                    pair = plsc.bitcast(word, jnp.bfloat16)
                    ev, od = plsc.unpack(pair,
                                         format=plsc.PackFormat.INTERLEAVED)
                    accs[wu % 4] = accs[wu % 4] + (
                        ev * rot(e_v, wu) + od * rot(o_v, wu))
            scores_v[pl.ds(ib + g * LANES, LANES)] = (
                (accs[0] + accs[1]) + (accs[2] + accs[3]))
            return 0

        lax.fori_loop(0, NGRP, g_step, 0)

    def chunk_step(ch, _):
        base = wbase + ch * CB
        pltpu.sync_copy(iword_hbm.at[pl.ds(base, CB)], iwv)
        pltpu.sync_copy(cwords_hbm.at[pl.ds(base * KPAD, CB * KPAD)], cidx_v)
        pltpu.async_copy(wi_hbm.at[iwv], ivecs_v, sem_i).wait()
        for s in range(3):
            issue4(s, s)

        def quad_step(q, _):
            b0 = 4 * q
            for j in range(4):
                bi = b0 + j
                nxt = bi + 3
                if j == 0:
                    issue4(nxt, (j + 3) % 4)
                else:
                    @pl.when(q < CB // 4 - 1)
                    def _():
                        issue4(nxt, (j + 3) % 4)
                wait4(bi, j)
                compute(bi, j)
            return 0

        lax.fori_loop(0, CB // 4, quad_step, 0)
        pltpu.sync_copy(scores_v, out_hbm.at[pl.ds(base * KPAD, CB * KPAD)])
        return 0

    lax.fori_loop(0, NCH, chunk_step, 0)


@functools.partial(
    pl.kernel,
    out_type=jax.ShapeDtypeStruct((B * KPAD,), jnp.float32),
    mesh=plsc.VectorSubcoreMesh(core_axis_name="c", subcore_axis_name="s",
                                num_cores=NC, num_subcores=NS),
    scratch_types=[
        pltpu.VMEM((CB,), jnp.int32),
        pltpu.VMEM((CB * KPAD,), jnp.int32),
        pltpu.VMEM((CB, D), jnp.float32),
        pltpu.VMEM((4, KPAD, WP), jnp.int32),
        pltpu.VMEM((CB * KPAD,), jnp.float32),
        pltpu.SemaphoreType.DMA,
        pltpu.SemaphoreType.DMA,
        pltpu.SemaphoreType.DMA,
        pltpu.SemaphoreType.DMA,
        pltpu.SemaphoreType.DMA,
    ],
    compiler_params=pltpu.CompilerParams(needs_layout_passes=False,
                                         use_tc_tiling_on_sc=False),
)
def _sc_scores(iword_hbm, cwords_hbm, wi_hbm, wo_hbm, out_hbm,
               iwv, cidx_v, ivecs_v, rows_v, scores_v,
               sem_i, sem0, sem1, sem2, sem3):
    _sc_scores_kernel(iword_hbm, cwords_hbm, wi_hbm, wo_hbm, out_hbm,
                      iwv, cidx_v, ivecs_v, rows_v, scores_v,
                      sem_i, sem0, sem1, sem2, sem3)


def _tc_loss_body(s_ref, o_ref):
    x = s_ref[...]
    col = lax.broadcasted_iota(jnp.int32, x.shape, 1)
    xs = jnp.where(col < C, x, -x)
    y = -jnp.logaddexp(0.0, -xs)
    y = jnp.where(col < K, y, 0.0)
    o_ref[0, 0] = -jnp.sum(y) / (B * C)


def kernel(iword, owords, nwords, Wi, Wo):
    pad = jnp.zeros((B, KPAD - K), jnp.int32)
    cwords = jnp.concatenate(
        [owords.astype(jnp.int32), nwords.astype(jnp.int32), pad], axis=1)
    wo_i32 = lax.bitcast_convert_type(
        Wo.astype(jnp.bfloat16).reshape(VOCAB, WP, 2), jnp.int32)
    scores_flat = _sc_scores(iword.astype(jnp.int32), cwords.reshape(-1),
                             Wi, wo_i32)
    scores = scores_flat.reshape(B, KPAD)
    loss = pl.pallas_call(
        _tc_loss_body,
        out_shape=jax.ShapeDtypeStruct((1, 1), jnp.float32),
        out_specs=pl.BlockSpec(memory_space=pltpu.SMEM),
    )(scores)
    return loss[0, 0]


# speedup vs baseline: 1.0752x; 1.0752x over previous
import functools

import jax
import jax.numpy as jnp
from jax import lax
from jax.experimental import pallas as pl
from jax.experimental.pallas import tpu as pltpu
from jax.experimental.pallas import tpu_sc as plsc

VOCAB = 100000
D = 64
B = 4096
C = 20
NNEG = 20

NC = 2
NS = 16
NW = NC * NS
LANES = 16

WP = D // 2
K = C + C * NNEG
KPAD = 432
NGRP = KPAD // LANES
B_PER_W = B // NW
CB = 16
NCH = B_PER_W // CB

_GATHER_SLICES = ((0, 128), (128, 128), (256, 128), (384, K - 384))


def _sc_scores_kernel(iword_hbm, cwords_hbm, wi_hbm, wo_hbm, out_hbm,
                      iwv, cidx_v, ivecs_v, rows_v, scores_v,
                      sem_i, sem0, sem1):
    wid = lax.axis_index("s") * NC + lax.axis_index("c")
    wbase = wid * B_PER_W
    slot_refs = (rows_v.at[0], rows_v.at[1])
    sems = (sem0, sem1)

    def issue4(bi, slot):
        ib = bi * KPAD
        for off, ln in _GATHER_SLICES:
            pltpu.async_copy(
                wo_hbm.at[cidx_v.at[pl.ds(ib + off, ln)]],
                slot_refs[slot].at[pl.ds(off, ln)], sems[slot])

    def wait4(bi, slot):
        ib = bi * KPAD
        for off, ln in _GATHER_SLICES:
            pltpu.make_async_copy(
                wo_hbm.at[cidx_v.at[pl.ds(ib + off, ln)]],
                slot_refs[slot].at[pl.ds(off, ln)], sems[slot]).wait()

    def compute(bi, slot):
        ib = bi * KPAD
        iota = lax.iota(jnp.int32, LANES)

        def rot(v, k):
            perm = (iota + k) & (LANES - 1)
            return v.at[perm].get(mode="promise_in_bounds")

        t_e = (2 * iota) & (LANES - 1)
        half = iota < (LANES // 2)
        eo = []
        for dd in range(WP // LANES):
            iv_a = ivecs_v[bi, pl.ds(dd * 2 * LANES, LANES)]
            iv_b = ivecs_v[bi, pl.ds(dd * 2 * LANES + LANES, LANES)]
            e_v = jnp.where(half,
                            iv_a.at[t_e].get(mode="promise_in_bounds"),
                            iv_b.at[t_e].get(mode="promise_in_bounds"))
            o_v = jnp.where(half,
                            iv_a.at[t_e + 1].get(mode="promise_in_bounds"),
                            iv_b.at[t_e + 1].get(mode="promise_in_bounds"))
            eo.append((e_v, o_v))

        def g_step(g, _):
            rid = g * LANES + iota
            zeros = jnp.zeros((LANES,), jnp.float32)
            accs = [zeros, zeros, zeros, zeros]
            for dd in range(WP // LANES):
                e_v, o_v = eo[dd]
                for wu in range(LANES):
                    perm = (iota + wu) & (LANES - 1)
                    word = plsc.load_gather(
                        slot_refs[slot], [rid, dd * LANES + perm])
                    pair = plsc.bitcast(word, jnp.bfloat16)
                    ev, od = plsc.unpack(pair,
                                         format=plsc.PackFormat.INTERLEAVED)
                    accs[wu % 4] = accs[wu % 4] + (
                        ev * rot(e_v, wu) + od * rot(o_v, wu))
            scores_v[pl.ds(ib + g * LANES, LANES)] = (
                (accs[0] + accs[1]) + (accs[2] + accs[3]))
            return 0

        lax.fori_loop(0, NGRP, g_step, 0)

    def chunk_step(ch, _):
        base = wbase + ch * CB
        pltpu.sync_copy(iword_hbm.at[pl.ds(base, CB)], iwv)
        pltpu.sync_copy(cwords_hbm.at[pl.ds(base * KPAD, CB * KPAD)], cidx_v)
        pltpu.async_copy(wi_hbm.at[iwv], ivecs_v, sem_i).wait()
        issue4(0, 0)

        def pair_step(p, _):
            b0 = 2 * p
            issue4(b0 + 1, 1)
            wait4(b0, 0)
            compute(b0, 0)

            @pl.when(p < CB // 2 - 1)
            def _():
                issue4(b0 + 2, 0)

            wait4(b0 + 1, 1)
            compute(b0 + 1, 1)
            return 0

        lax.fori_loop(0, CB // 2, pair_step, 0)
        pltpu.sync_copy(scores_v, out_hbm.at[pl.ds(base * KPAD, CB * KPAD)])
        return 0

    lax.fori_loop(0, NCH, chunk_step, 0)


@functools.partial(
    pl.kernel,
    out_type=jax.ShapeDtypeStruct((B * KPAD,), jnp.float32),
    mesh=plsc.VectorSubcoreMesh(core_axis_name="c", subcore_axis_name="s",
                                num_cores=NC, num_subcores=NS),
    scratch_types=[
        pltpu.VMEM((CB,), jnp.int32),
        pltpu.VMEM((CB * KPAD,), jnp.int32),
        pltpu.VMEM((CB, D), jnp.float32),
        pltpu.VMEM((2, KPAD, WP), jnp.int32),
        pltpu.VMEM((CB * KPAD,), jnp.float32),
        pltpu.SemaphoreType.DMA,
        pltpu.SemaphoreType.DMA,
        pltpu.SemaphoreType.DMA,
    ],
    compiler_params=pltpu.CompilerParams(needs_layout_passes=False,
                                         use_tc_tiling_on_sc=False),
)
def _sc_scores(iword_hbm, cwords_hbm, wi_hbm, wo_hbm, out_hbm,
               iwv, cidx_v, ivecs_v, rows_v, scores_v, sem_i, sem0, sem1):
    _sc_scores_kernel(iword_hbm, cwords_hbm, wi_hbm, wo_hbm, out_hbm,
                      iwv, cidx_v, ivecs_v, rows_v, scores_v,
                      sem_i, sem0, sem1)


def _tc_loss_body(s_ref, o_ref):
    x = s_ref[...]
    col = lax.broadcasted_iota(jnp.int32, x.shape, 1)
    xs = jnp.where(col < C, x, -x)
    y = -jnp.logaddexp(0.0, -xs)
    y = jnp.where(col < K, y, 0.0)
    o_ref[0, 0] = -jnp.sum(y) / (B * C)


def kernel(iword, owords, nwords, Wi, Wo):
    pad = jnp.zeros((B, KPAD - K), jnp.int32)
    cwords = jnp.concatenate(
        [owords.astype(jnp.int32), nwords.astype(jnp.int32), pad], axis=1)
    wo_i32 = lax.bitcast_convert_type(
        Wo.astype(jnp.bfloat16).reshape(VOCAB, WP, 2), jnp.int32)
    scores_flat = _sc_scores(iword.astype(jnp.int32), cwords.reshape(-1),
                             Wi, wo_i32)
    scores = scores_flat.reshape(B, KPAD)
    loss = pl.pallas_call(
        _tc_loss_body,
        out_shape=jax.ShapeDtypeStruct((1, 1), jnp.float32),
        out_specs=pl.BlockSpec(memory_space=pltpu.SMEM),
    )(scores)
    return loss[0, 0]
